# baseline (speedup 1.0000x reference)
import functools

import jax
import jax.numpy as jnp
from jax import lax
from jax.experimental import pallas as pl
from jax.experimental.pallas import tpu as pltpu
from jax.experimental.pallas import tpu_sc as plsc

_E = 320000
_N = 10000
_NU = 128
_NB = 16

_NC = 2
_NS = 16
_NW = _NC * _NS
_EPW = _E // _NW

_BN = 2000
_NBLK = _N // _BN


def _sc_scatter_body(idx1, idx2, vx, vy, part,
                     i1_v, i2_v, vx_v, vy_v, zbuf, acc, lsem, ssem):
    c = lax.axis_index("c")
    s = lax.axis_index("s")
    w = s * _NC + c
    base = w * _EPW

    @pl.when(s < _NBLK)
    def _zero():
        def zstore(i, carry):
            zbuf[pl.ds(i * 16, 16)] = jnp.zeros((16,), jnp.float32)
            return carry

        lax.fori_loop(0, _BN // 16, zstore, 0)
        pltpu.sync_copy(zbuf, acc.at[pl.ds(s * _BN, _BN)])

    loads = [
        pltpu.async_copy(idx1.at[pl.ds(base, _EPW)], i1_v, lsem),
        pltpu.async_copy(idx2.at[pl.ds(base, _EPW)], i2_v, lsem),
        pltpu.async_copy(vx.at[pl.ds(base, _EPW)], vx_v, lsem),
        pltpu.async_copy(vy.at[pl.ds(base, _EPW)], vy_v, lsem),
    ]
    for d in loads:
        d.wait()

    plsc.subcore_barrier()

    d1 = pltpu.async_copy(vx_v, acc.at[i1_v], ssem, add=True)
    d2 = pltpu.async_copy(vy_v, acc.at[i2_v], ssem, add=True)
    d1.wait()
    d2.wait()

    plsc.subcore_barrier()

    @pl.when(s < _NBLK)
    def _flush():
        pltpu.sync_copy(acc.at[pl.ds(s * _BN, _BN)], part.at[c, s, 0])


_sc_scatter = functools.partial(
    pl.kernel,
    out_type=jax.ShapeDtypeStruct((_NC, _NBLK, 1, _BN), jnp.float32),
    mesh=plsc.VectorSubcoreMesh(core_axis_name="c", subcore_axis_name="s",
                                num_cores=_NC, num_subcores=_NS),
    scratch_types=[
        pltpu.VMEM((_EPW,), jnp.int32),
        pltpu.VMEM((_EPW,), jnp.int32),
        pltpu.VMEM((_EPW,), jnp.float32),
        pltpu.VMEM((_EPW,), jnp.float32),
        pltpu.VMEM((_BN,), jnp.float32),
        pltpu.VMEM_SHARED((_N,), jnp.float32),
        pltpu.SemaphoreType.DMA,
        pltpu.SemaphoreType.DMA,
    ],
    compiler_params=pltpu.CompilerParams(use_tc_tiling_on_sc=False),
)(_sc_scatter_body)


def _tc_assemble_body(part_ref, out_ref):
    onehot = (lax.broadcasted_iota(jnp.int32, (_NC, _NU), 1) == 0)
    out_ref[...] = lax.dot_general(
        part_ref[:, 0, 0, :],
        onehot.astype(jnp.float32),
        (((0,), (0,)), ((), ())),
        preferred_element_type=jnp.float32,
        precision=lax.Precision.HIGHEST,
    )


def kernel(unary, binary, deltas, index1, index2):
    del unary, binary

    i1 = index1.astype(jnp.int32)
    i2 = index2.astype(jnp.int32)
    vx = deltas[:, 0]
    vy = deltas[:, _NU]

    parts = _sc_scatter(i1, i2, vx, vy)

    out1 = pl.pallas_call(
        _tc_assemble_body,
        grid=(_NBLK,),
        in_specs=[pl.BlockSpec((_NC, 1, 1, _BN), lambda i: (0, i, 0, 0))],
        out_specs=pl.BlockSpec((_BN, _NU), lambda i: (i, 0)),
        out_shape=jax.ShapeDtypeStruct((_N, _NU), jnp.float32),
    )(parts)

    return (out1, deltas[:, 2 * _NU:])


# speedup vs baseline: 1.0417x; 1.0417x over previous
import functools

import jax
import jax.numpy as jnp
from jax import lax
from jax.experimental import pallas as pl
from jax.experimental.pallas import tpu as pltpu
from jax.experimental.pallas import tpu_sc as plsc

_E = 320000
_N = 10000
_NU = 128
_NB = 16

_NC = 2
_NS = 16
_NW = _NC * _NS
_EPW = _E // _NW

_BN = 2000
_NBLK = _N // _BN


def _sc_scatter_body(idx1, idx2, vx, vy, part,
                     i1_v, i2_v, vx_v, vy_v, zbuf, acc, lsem, ssem):
    c = lax.axis_index("c")
    s = lax.axis_index("s")
    w = s * _NC + c
    base = w * _EPW

    @pl.when(s < _NBLK)
    def _zero():
        def zstore(i, carry):
            zbuf[pl.ds(i * 16, 16)] = jnp.zeros((16,), jnp.float32)
            return carry

        lax.fori_loop(0, _BN // 16, zstore, 0)
        pltpu.sync_copy(zbuf, acc.at[pl.ds(s * _BN, _BN)])

    loads = [
        pltpu.async_copy(idx1.at[pl.ds(base, _EPW)], i1_v, lsem),
        pltpu.async_copy(idx2.at[pl.ds(base, _EPW)], i2_v, lsem),
        pltpu.async_copy(vx.at[pl.ds(base, _EPW)], vx_v, lsem),
        pltpu.async_copy(vy.at[pl.ds(base, _EPW)], vy_v, lsem),
    ]
    for d in loads:
        d.wait()

    plsc.subcore_barrier()

    d1 = pltpu.async_copy(vx_v, acc.at[i1_v], ssem, add=True)
    d2 = pltpu.async_copy(vy_v, acc.at[i2_v], ssem, add=True)
    d1.wait()
    d2.wait()

    plsc.subcore_barrier()

    @pl.when(s < _NBLK)
    def _flush():
        pltpu.sync_copy(acc.at[pl.ds(s * _BN, _BN)], part.at[c, s, 0])


_sc_scatter = functools.partial(
    pl.kernel,
    out_type=jax.ShapeDtypeStruct((_NC, _NBLK, 1, _BN), jnp.float32),
    mesh=plsc.VectorSubcoreMesh(core_axis_name="c", subcore_axis_name="s",
                                num_cores=_NC, num_subcores=_NS),
    scratch_types=[
        pltpu.VMEM((_EPW,), jnp.int32),
        pltpu.VMEM((_EPW,), jnp.int32),
        pltpu.VMEM((_EPW,), jnp.float32),
        pltpu.VMEM((_EPW,), jnp.float32),
        pltpu.VMEM((_BN,), jnp.float32),
        pltpu.VMEM_SHARED((_N,), jnp.float32),
        pltpu.SemaphoreType.DMA,
        pltpu.SemaphoreType.DMA,
    ],
    compiler_params=pltpu.CompilerParams(use_tc_tiling_on_sc=False),
)(_sc_scatter_body)


def _tc_assemble_body(part_ref, out_ref):
    p = part_ref[0, 0, 0, :] + part_ref[1, 0, 0, :]
    lanes = lax.broadcasted_iota(jnp.int32, (_BN, _NU), 1)
    out_ref[...] = jnp.where(lanes == 0, p[:, None], jnp.float32(0.0))


def kernel(unary, binary, deltas, index1, index2):
    del unary, binary

    i1 = index1.astype(jnp.int32)
    i2 = index2.astype(jnp.int32)
    vx = deltas[:, 0]
    vy = deltas[:, _NU]

    parts = _sc_scatter(i1, i2, vx, vy)

    out1 = pl.pallas_call(
        _tc_assemble_body,
        grid=(_NBLK,),
        in_specs=[pl.BlockSpec((_NC, 1, 1, _BN), lambda i: (0, i, 0, 0))],
        out_specs=pl.BlockSpec((_BN, _NU), lambda i: (i, 0)),
        out_shape=jax.ShapeDtypeStruct((_N, _NU), jnp.float32),
    )(parts)

    return (out1, deltas[:, 2 * _NU:])
